# Initial kernel scaffold; baseline (speedup 1.0000x reference)
import jax
import jax.numpy as jnp
from jax.experimental import pallas as pl


def kernel(node_features, edge_features, memory, source_nodes, timestamps, neighbors, edge_idxs, edge_times, time_w, time_b, Wq, Wk, Wv, fc1_w, fc1_b, fc2_w, fc2_b):
    raise NotImplementedError("write your pallas kernel here")


# speedup vs baseline: 5.2078x; 5.2078x over previous
import functools

import jax
import jax.numpy as jnp
from jax import lax
from jax.experimental import pallas as pl
from jax.experimental.pallas import tpu as pltpu
from jax.experimental.pallas import tpu_sc as plsc

_F32 = jnp.float32

_B = 16384
_NGH = 20
_D = 128
_DE = 16
_DT = 128
_H = 2
_DH = 64

_NW = 32
_GW = 128
_BB = 256


def _combine_body(nf_ref, mem_ref, out_ref):
    out_ref[...] = nf_ref[...] + mem_ref[...]


def _combined_table(node_features, memory):
    n, d = node_features.shape
    blk = 2000 if n % 2000 == 0 else n
    return pl.pallas_call(
        _combine_body,
        out_shape=jax.ShapeDtypeStruct((n, d), _F32),
        grid=(n // blk,),
        in_specs=[pl.BlockSpec((blk, d), lambda i: (i, 0))] * 2,
        out_specs=pl.BlockSpec((blk, d), lambda i: (i, 0)),
    )(node_features, memory)


def _sc_gather(combined, etab8, ngh_idx, src_idx, egrp_idx):
    n_ngh = ngh_idx.shape[0]
    n_src = src_idx.shape[0]
    per_w = n_ngh // _NW
    per_w_src = n_src // _NW
    mesh = plsc.VectorSubcoreMesh(core_axis_name="c", subcore_axis_name="s")

    @functools.partial(
        pl.kernel,
        out_type=[
            jax.ShapeDtypeStruct((n_ngh, _D), _F32),
            jax.ShapeDtypeStruct((n_src, _D), _F32),
            jax.ShapeDtypeStruct((n_ngh, _D), _F32),
        ],
        mesh=mesh,
        scratch_types=[
            pltpu.VMEM((per_w,), jnp.int32),
            pltpu.VMEM((_GW, _D), _F32),
            pltpu.SemaphoreType.DMA,
        ],
    )
    def gather_kernel(tab_hbm, etab_hbm, ngh_idx_hbm, src_idx_hbm, edge_idx_hbm,
                      ngh_out, src_out, edge_out, idx_v, rows_v, sem):
        wid = lax.axis_index("s") * 2 + lax.axis_index("c")

        base = wid * per_w
        pltpu.sync_copy(ngh_idx_hbm.at[pl.ds(base, per_w)], idx_v)

        @pl.loop(0, per_w // _GW)
        def _(c):
            off = c * _GW
            pltpu.async_copy(
                tab_hbm.at[idx_v.at[pl.ds(off, _GW)]], rows_v, sem).wait()
            pltpu.sync_copy(rows_v, ngh_out.at[pl.ds(base + off, _GW)])

        pltpu.sync_copy(edge_idx_hbm.at[pl.ds(base, per_w)], idx_v)

        @pl.loop(0, per_w // _GW)
        def _(c):
            off = c * _GW
            pltpu.async_copy(
                etab_hbm.at[idx_v.at[pl.ds(off, _GW)]], rows_v, sem).wait()
            pltpu.sync_copy(rows_v, edge_out.at[pl.ds(base + off, _GW)])

        sbase = wid * per_w_src
        pltpu.sync_copy(src_idx_hbm.at[pl.ds(sbase, per_w_src)],
                        idx_v.at[pl.ds(0, per_w_src)])

        @pl.loop(0, per_w_src // _GW)
        def _(c):
            off = c * _GW
            pltpu.async_copy(
                tab_hbm.at[idx_v.at[pl.ds(off, _GW)]], rows_v, sem).wait()
            pltpu.sync_copy(rows_v, src_out.at[pl.ds(sbase + off, _GW)])

    return gather_kernel(combined, etab8, ngh_idx, src_idx, egrp_idx)


def _main_body(src_ref, ngh_ref, egrp_ref, erem_ref, ts_ref, et_ref, ei_ref,
               tw_ref, tb_ref, wq_ref, wk_ref, wv_ref, f1w_ref, f1b_ref,
               f2w_ref, f2b_ref, out_ref):
    src = src_ref[...]
    tb = tb_ref[...]
    tw = tw_ref[...]

    qbias = jnp.dot(jnp.cos(tb), wq_ref[_D:, :], preferred_element_type=_F32)
    q = jnp.dot(src, wq_ref[:_D, :], preferred_element_type=_F32) + qbias

    delta = ts_ref[...] - et_ref[...]
    t3 = jnp.cos(delta * tw[None] + tb[None])

    ngh2 = ngh_ref[...].reshape(_NGH * _BB, _D)
    t2 = t3.reshape(_NGH * _BB, _DT)

    grp = egrp_ref[...]
    rem = erem_ref[...]
    e3 = grp[:, :, :_DE]
    for rr in range(1, 8):
        e3 = jnp.where(rem == rr, grp[:, :, rr * _DE:(rr + 1) * _DE], e3)
    edge2 = e3.reshape(_NGH * _BB, _DE)

    k2 = (jnp.dot(ngh2, wk_ref[:_D, :], preferred_element_type=_F32)
          + jnp.dot(t2, wk_ref[_D:_D + _DT, :], preferred_element_type=_F32)
          + jnp.dot(edge2, wk_ref[_D + _DT:, :], preferred_element_type=_F32))
    v2 = (jnp.dot(ngh2, wv_ref[:_D, :], preferred_element_type=_F32)
          + jnp.dot(t2, wv_ref[_D:_D + _DT, :], preferred_element_type=_F32)
          + jnp.dot(edge2, wv_ref[_D + _DT:, :], preferred_element_type=_F32))
    k3 = k2.reshape(_NGH, _BB, _H * _DH)
    v3 = v2.reshape(_NGH, _BB, _H * _DH)

    prod = k3 * q[None]
    s0 = jnp.sum(prod[:, :, :_DH], axis=-1, keepdims=True)
    s1 = jnp.sum(prod[:, :, _DH:], axis=-1, keepdims=True)
    mask = jnp.where(ei_ref[...] == 0, _F32(-1e10), _F32(0.0))
    s0 = s0 * _F32(0.125) + mask
    s1 = s1 * _F32(0.125) + mask

    e0 = jnp.exp(s0 - jnp.max(s0, axis=0, keepdims=True))
    a0 = e0 / jnp.sum(e0, axis=0, keepdims=True)
    e1 = jnp.exp(s1 - jnp.max(s1, axis=0, keepdims=True))
    a1 = e1 / jnp.sum(e1, axis=0, keepdims=True)

    o0 = jnp.sum(v3[:, :, :_DH] * a0, axis=0)
    o1 = jnp.sum(v3[:, :, _DH:] * a1, axis=0)
    attn = jnp.concatenate([o0, o1], axis=-1)

    h1 = (jnp.dot(attn, f1w_ref[:_H * _DH, :], preferred_element_type=_F32)
          + jnp.dot(src, f1w_ref[_H * _DH:, :], preferred_element_type=_F32)
          + f1b_ref[...])
    h1 = jnp.maximum(h1, _F32(0.0))
    out_ref[...] = (jnp.dot(h1, f2w_ref[...], preferred_element_type=_F32)
                    + f2b_ref[...])


def _tc_main(src_rows, ngh3, egrp3, erem3, ts3, et3, ei3, tw, tb, Wq, Wk, Wv,
             fc1_w, fc1_b, fc2_w, fc2_b):
    grid = (_B // _BB,)
    return pl.pallas_call(
        _main_body,
        out_shape=jax.ShapeDtypeStruct((_B, _D), _F32),
        grid=grid,
        in_specs=[
            pl.BlockSpec((_BB, _D), lambda i: (i, 0)),
            pl.BlockSpec((_NGH, _BB, _D), lambda i: (0, i, 0)),
            pl.BlockSpec((_NGH, _BB, _D), lambda i: (0, i, 0)),
            pl.BlockSpec((_NGH, _BB, 1), lambda i: (0, i, 0)),
            pl.BlockSpec((1, _BB, 1), lambda i: (0, i, 0)),
            pl.BlockSpec((_NGH, _BB, 1), lambda i: (0, i, 0)),
            pl.BlockSpec((_NGH, _BB, 1), lambda i: (0, i, 0)),
            pl.BlockSpec((1, _DT), lambda i: (0, 0)),
            pl.BlockSpec((1, _DT), lambda i: (0, 0)),
            pl.BlockSpec((_D + _DT, _H * _DH), lambda i: (0, 0)),
            pl.BlockSpec((_D + _DT + _DE, _H * _DH), lambda i: (0, 0)),
            pl.BlockSpec((_D + _DT + _DE, _H * _DH), lambda i: (0, 0)),
            pl.BlockSpec((_H * _DH + _D, _D), lambda i: (0, 0)),
            pl.BlockSpec((1, _D), lambda i: (0, 0)),
            pl.BlockSpec((_D, _D), lambda i: (0, 0)),
            pl.BlockSpec((1, _D), lambda i: (0, 0)),
        ],
        out_specs=pl.BlockSpec((_BB, _D), lambda i: (i, 0)),
    )(src_rows, ngh3, egrp3, erem3, ts3, et3, ei3, tw, tb, Wq, Wk, Wv,
      fc1_w, fc1_b, fc2_w, fc2_b)


def kernel(node_features, edge_features, memory, source_nodes, timestamps,
           neighbors, edge_idxs, edge_times, time_w, time_b, Wq, Wk, Wv,
           fc1_w, fc1_b, fc2_w, fc2_b):
    assert node_features.shape == (10000, _D)
    assert neighbors.shape == (_B, _NGH)
    assert edge_features.shape[1] == _DE

    combined = _combined_table(node_features, memory)
    etab8 = edge_features.reshape(edge_features.shape[0] // 8, _D)

    ngh_idx = neighbors.T.reshape(-1).astype(jnp.int32)
    edge_idx = edge_idxs.T.astype(jnp.int32)
    src_idx = source_nodes.astype(jnp.int32)

    ngh_rows, src_rows, edge_rows = _sc_gather(
        combined, etab8, ngh_idx, src_idx, (edge_idx // 8).reshape(-1))

    ngh3 = ngh_rows.reshape(_NGH, _B, _D)
    egrp3 = edge_rows.reshape(_NGH, _B, _D)
    erem3 = (edge_idx % 8).reshape(_NGH, _B, 1)
    ts3 = timestamps.astype(_F32).reshape(1, _B, 1)
    et3 = edge_times.T.astype(_F32).reshape(_NGH, _B, 1)
    ei3 = edge_idx.reshape(_NGH, _B, 1)

    return _tc_main(
        src_rows, ngh3, egrp3, erem3, ts3, et3, ei3,
        time_w.reshape(1, _DT), time_b.reshape(1, _DT), Wq, Wk, Wv,
        fc1_w, fc1_b.reshape(1, _D), fc2_w, fc2_b.reshape(1, _D))
